# baseline (speedup 1.0000x reference)
import functools

import jax
import jax.numpy as jnp
from jax import lax
from jax.experimental import pallas as pl
from jax.experimental.pallas import tpu as pltpu
from jax.experimental.pallas import tpu_sc as plsc

_L = 16
_B = 16384
_C = 5
_V = 1000000
_D = 64
_VT = (_V // 128) * 128
_SW = 512
_NSL = _VT // _SW
_NW = 32
_CAP = 3072
_NIT = _B + _B * _C
_DUMP = _NIT
_SCR = _NIT + 16


def _make_extract_kernel():
    mesh = plsc.VectorSubcoreMesh(core_axis_name="c", subcore_axis_name="s")

    @functools.partial(
        pl.kernel,
        mesh=mesh,
        compiler_params=pltpu.CompilerParams(
            needs_layout_passes=False, use_tc_tiling_on_sc=True),
        out_type=jax.ShapeDtypeStruct((_SCR, 128), jnp.float32),
        scratch_types=[
            pltpu.VMEM((2048,), jnp.int32),
            pltpu.VMEM((_CAP,), jnp.int32),
            pltpu.VMEM((_CAP,), jnp.int32),
            pltpu.VMEM((64, _SW), jnp.float32),
            pltpu.VMEM((64, _SW), jnp.float32),
            pltpu.VMEM((64, 64), jnp.float32),
            pltpu.VMEM((64, 64), jnp.float32),
            pltpu.VMEM((_CAP,), jnp.int32),
            pltpu.VMEM((_CAP,), jnp.int32),
            pltpu.VMEM((64 * 17,), jnp.float32),
            pltpu.VMEM((64, 128), jnp.float32),
            pltpu.VMEM((64,), jnp.int32),
            pltpu.SemaphoreType.DMA,
            pltpu.SemaphoreType.DMA,
            pltpu.SemaphoreType.DMA,
        ],
    )
    def ka(tgt_hbm, ctx_hbm, ttabT, ctabT, tail_t_hbm, tail_c_hbm, scr_hbm,
           idxb, wl_v, wl_s, sl_t, sl_c, tt_v, tc_v, ml_v, ml_s,
           obt, obuf, grp, sem_a, sem_b, sem_o):
        wid = lax.axis_index("s") * 2 + lax.axis_index("c")
        lanes = lax.iota(jnp.int32, _L)
        s0 = (_NSL * wid) // _NW
        s1 = (_NSL * (wid + 1)) // _NW
        lo = s0 * _SW
        hi = jnp.where(wid == _NW - 1, _V, s1 * _SW)

        pltpu.sync_copy(tail_t_hbm, tt_v)
        pltpu.sync_copy(tail_c_hbm, tc_v)

        def scan_arr(src_ref, n_items, slot_off, skip, carry):
            def ch_body(cb, car):
                pltpu.sync_copy(src_ref.at[pl.ds(cb * 2048, 2048)], idxb)

                def jb(j, car2):
                    cur2, seen2 = car2
                    v = idxb[pl.ds(j * _L, _L)]
                    pos = cb * 2048 + j * _L + slot_off + lanes
                    m = (v >= lo) & (v < hi)
                    mi = jnp.where(m, 1, 0).astype(jnp.int32)
                    r = plsc.cumsum(mi)
                    grank = seen2 + r - 1
                    acc = m & (grank >= skip) & (grank < skip + _CAP)
                    ai = jnp.where(acc, 1, 0).astype(jnp.int32)
                    ar = plsc.cumsum(ai) - 1
                    plsc.store_scatter(wl_v, [cur2 + ar], v, mask=acc)
                    plsc.store_scatter(wl_s, [cur2 + ar], pos, mask=acc)
                    return (cur2 + jnp.sum(ai), seen2 + jnp.sum(mi))

                return lax.fori_loop(0, 128, jb, car)

            return lax.fori_loop(0, n_items // 2048, ch_body, carry)

        def process_slice(st_ref, sc_ref, v0, ws, nwl):
            def mt(j, mc):
                v = wl_v[pl.ds(j * _L, _L)]
                valid = (j * _L + lanes) < nwl
                ok = valid & (v >= v0) & (v < v0 + ws)
                oi = jnp.where(ok, 1, 0).astype(jnp.int32)
                oe = plsc.cumsum(oi) - 1
                plsc.store_scatter(ml_v, [mc + oe], v - v0, mask=ok)
                plsc.store_scatter(
                    ml_s, [mc + oe], wl_s[pl.ds(j * _L, _L)], mask=ok)
                return mc + jnp.sum(oi)

            mcur = lax.fori_loop(0, _CAP // _L, mt, 0)
            padm = lanes < ((_L - (mcur & (_L - 1))) & (_L - 1))
            plsc.store_scatter(
                ml_v, [mcur + lanes], jnp.zeros((_L,), jnp.int32), mask=padm)
            plsc.store_scatter(
                ml_s, [mcur + lanes],
                jnp.full((_L,), _DUMP, jnp.int32), mask=padm)
            ngrp = (mcur + _L - 1) // _L

            def bt(t, btcar):
                for b8 in range(4):
                    g = t * 4 + b8
                    grp[pl.ds(b8 * _L, _L)] = jnp.full(
                        (_L,), _DUMP, jnp.int32)

                    @pl.when(g < ngrp)
                    def _():
                        vloc = ml_v[pl.ds(g * _L, _L)]
                        slot = ml_s[pl.ds(g * _L, _L)]
                        ist = slot < _B
                        for d in range(64):
                            dv = jnp.full((_L,), d, jnp.int32)
                            va = plsc.load_gather(st_ref, [dv, vloc])
                            vb = plsc.load_gather(sc_ref, [dv, vloc])
                            vals = jnp.where(ist, va, vb)
                            plsc.store_scatter(obt, [lanes + d * 17], vals)
                        for i in range(_L):
                            for kk in range(4):
                                rv = plsc.load_gather(
                                    obt, [(lanes + kk * _L) * 17 + i])
                                obuf[b8 * _L + i, pl.ds(kk * _L, _L)] = rv
                        grp[pl.ds(b8 * _L, _L)] = slot

                pltpu.async_copy(obuf, scr_hbm.at[grp], sem_o).wait()
                return btcar

            lax.fori_loop(0, (ngrp + 3) // 4, bt, 0)

        def pass_body(state):
            skip, _total = state
            cur, seen = scan_arr(
                tgt_hbm, _B, 0, skip, (jnp.int32(0), jnp.int32(0)))
            nwl, total2 = scan_arr(ctx_hbm, _B * _C, _B, skip, (cur, seen))

            def sl_body(s, _):
                v0 = pl.multiple_of(s * _SW, 128)
                d1 = pltpu.async_copy(
                    ttabT.at[:, pl.ds(v0, _SW)], sl_t, sem_a)
                d2 = pltpu.async_copy(
                    ctabT.at[:, pl.ds(v0, _SW)], sl_c, sem_b)
                d1.wait()
                d2.wait()
                process_slice(sl_t, sl_c, v0, _SW, nwl)
                return _

            lax.fori_loop(s0, s1, sl_body, 0)

            @pl.when(wid == _NW - 1)
            def _():
                process_slice(tt_v, tc_v, _VT, _V - _VT, nwl)

            return (skip + _CAP, total2)

        lax.while_loop(
            lambda st: st[0] < st[1], pass_body,
            (jnp.int32(0), jnp.int32(1)))

    return ka


def _make_dot_kernel():
    mesh = plsc.VectorSubcoreMesh(core_axis_name="c", subcore_axis_name="s")
    CB = 64
    NCH = (_B // _NW) // CB

    @functools.partial(
        pl.kernel,
        mesh=mesh,
        compiler_params=pltpu.CompilerParams(
            needs_layout_passes=False, use_tc_tiling_on_sc=False),
        out_type=jax.ShapeDtypeStruct((_B * _C,), jnp.float32),
        scratch_types=[
            pltpu.VMEM((CB, 128), jnp.float32),
            pltpu.VMEM((CB * _C, 128), jnp.float32),
            pltpu.VMEM((80 * _L,), jnp.float32),
            pltpu.VMEM((CB * _C,), jnp.float32),
        ],
    )
    def kb(scr_hbm, out_hbm, we_v, ce_v, acc_buf, out_v):
        wid = lax.axis_index("s") * 2 + lax.axis_index("c")
        lanes = lax.iota(jnp.int32, _L)

        def chunk_body(ch, carry):
            b_base = pl.multiple_of(wid * (_B // _NW) + ch * CB, CB)
            pltpu.sync_copy(scr_hbm.at[pl.ds(b_base, CB)], we_v)
            pltpu.sync_copy(
                scr_hbm.at[pl.ds(
                    pl.multiple_of(_B + b_base * _C, CB * _C), CB * _C)],
                ce_v)

            def g_body(g, gcarry):
                for i in range(_L):
                    b = g * _L + i
                    wv = [we_v[b, pl.ds(kk * _L, _L)] for kk in range(4)]
                    for c in range(_C):
                        q = i * _C + c
                        acc = wv[0] * ce_v[b * _C + c, pl.ds(0, _L)]
                        for kk in range(1, 4):
                            acc = acc + wv[kk] * ce_v[
                                b * _C + c, pl.ds(kk * _L, _L)]
                        acc_buf[pl.ds(q * _L, _L)] = acc
                for t in range(_C):
                    base_idx = (lanes + t * _L) * _L
                    out_vec = plsc.load_gather(acc_buf, [base_idx])
                    for j in range(1, _L):
                        col = plsc.load_gather(acc_buf, [base_idx + j])
                        out_vec = out_vec + col
                    out_v[pl.ds(g * 80 + t * _L, _L)] = out_vec
                return gcarry

            lax.fori_loop(0, CB // _L, g_body, 0)
            pltpu.sync_copy(
                out_v, out_hbm.at[pl.ds(b_base * _C, CB * _C)])
            return carry

        lax.fori_loop(0, NCH, chunk_body, 0)

    return kb


def kernel(target, context, target_table, context_table):
    B, C = context.shape
    tgt_flat = target.reshape(B)
    ctx_flat = context.reshape(B * C)
    ka = _make_extract_kernel()
    kb = _make_dot_kernel()
    scratch = ka(tgt_flat, ctx_flat,
                 target_table.T, context_table.T,
                 target_table[_VT:].T, context_table[_VT:].T)
    out_flat = kb(scratch)
    return out_flat.reshape(B, C)


# speedup vs baseline: 2.9579x; 2.9579x over previous
import functools

import jax
import jax.numpy as jnp
from jax import lax
from jax.experimental import pallas as pl
from jax.experimental.pallas import tpu as pltpu
from jax.experimental.pallas import tpu_sc as plsc

_L = 16


def _make_sc_kernel(B, C, D, V):
    NW = 32
    BPW = B // NW
    CB = 128
    NCH = BPW // CB
    NG = CB // _L
    IW = 80
    NJ = (CB * C) // IW
    KD = D // _L

    mesh = plsc.VectorSubcoreMesh(core_axis_name="c", subcore_axis_name="s")

    @functools.partial(
        pl.kernel,
        mesh=mesh,
        compiler_params=pltpu.CompilerParams(
            needs_layout_passes=False, use_tc_tiling_on_sc=False),
        out_type=jax.ShapeDtypeStruct((B * C,), jnp.float32),
        scratch_types=[
            pltpu.VMEM((CB,), jnp.int32),
            pltpu.VMEM((NJ, IW), jnp.int32),
            pltpu.VMEM((CB, D), jnp.float32),
            pltpu.VMEM((NJ, IW, D), jnp.float32),
            pltpu.VMEM((IW * _L,), jnp.float32),
            pltpu.VMEM((CB * C,), jnp.float32),
            pltpu.SemaphoreType.DMA,
            pltpu.SemaphoreType.DMA,
        ],
    )
    def k(tgt_hbm, ctx_hbm, ttab_hbm, ctab_hbm, out_hbm,
          tgt_idx, ce_idx, we_v, ce_v, acc_buf, out_v, sem_t, sem_c):
        wid = lax.axis_index("s") * 2 + lax.axis_index("c")
        lanes = lax.iota(jnp.int32, _L)

        def chunk_body(ch, carry):
            b_base = pl.multiple_of(wid * BPW + ch * CB, CB)
            pltpu.sync_copy(tgt_hbm.at[pl.ds(b_base, CB)], tgt_idx)
            pltpu.sync_copy(
                ctx_hbm.at[pl.ds(pl.multiple_of((b_base * C) // IW, NJ), NJ)],
                ce_idx)
            dwe = pltpu.async_copy(ttab_hbm.at[tgt_idx], we_v, sem_t)
            dce = [
                pltpu.async_copy(ctab_hbm.at[ce_idx.at[j]], ce_v.at[j], sem_c)
                for j in range(NJ)
            ]
            dwe.wait()
            for d in dce:
                d.wait()

            def g_body(g, gcarry):
                for i in range(_L):
                    b = g * _L + i
                    wv = [we_v[b, pl.ds(kk * _L, _L)] for kk in range(KD)]
                    for c in range(C):
                        q = i * C + c
                        acc = wv[0] * ce_v[g, q, pl.ds(0, _L)]
                        for kk in range(1, KD):
                            acc = acc + wv[kk] * ce_v[g, q, pl.ds(kk * _L, _L)]
                        acc_buf[pl.ds(q * _L, _L)] = acc
                for t in range(C):
                    base_idx = (lanes + t * _L) * _L
                    out_vec = plsc.load_gather(acc_buf, [base_idx])
                    for j in range(1, _L):
                        col = plsc.load_gather(acc_buf, [base_idx + j])
                        out_vec = out_vec + col
                    out_v[pl.ds(g * IW + t * _L, _L)] = out_vec
                return gcarry

            lax.fori_loop(0, NG, g_body, 0)
            pltpu.sync_copy(out_v, out_hbm.at[pl.ds(b_base * C, CB * C)])
            return carry

        lax.fori_loop(0, NCH, chunk_body, 0)

    return k


def kernel(target, context, target_table, context_table):
    B, C = context.shape
    V, D = target_table.shape
    tgt_flat = target.reshape(B)
    ctx_2d = context.reshape((B * C) // 80, 80)
    k = _make_sc_kernel(B, C, D, V)
    out_flat = k(tgt_flat, ctx_2d, target_table, context_table)
    return out_flat.reshape(B, C)
